# baseline (speedup 1.0000x reference)
import functools

import jax
import jax.numpy as jnp
from jax import lax
from jax.experimental import pallas as pl
from jax.experimental.pallas import tpu as pltpu
from jax.experimental.pallas import tpu_sc as plsc

N = 10000
K = 32
D = 128
NW = 32
NP = 10240
BPW = NP // NW
C = 1
CK = C * K
NCH = BPW // C
NBUF = 8
LANES = 16
NVEC = D // LANES


def _sc_body(x_hbm, nbr_hbm, out_hbm, idx_v, buf0, buf1, buf2, buf3, buf4, buf5, buf6, buf7, out_v, sem0, sem1, sem2, sem3, sem4, sem5, sem6, sem7):
    cid = lax.axis_index("c")
    sid = lax.axis_index("s")
    wid = sid * 2 + cid
    pltpu.sync_copy(nbr_hbm.at[pl.ds(wid * NCH, NCH)], idx_v)

    bufs = (buf0, buf1, buf2, buf3, buf4, buf5, buf6, buf7)
    sems = (sem0, sem1, sem2, sem3, sem4, sem5, sem6, sem7)

    def start(i, buf, sem):
        pltpu.async_copy(x_hbm.at[idx_v.at[i]], buf, sem)

    def wait(buf, sem):
        pltpu.make_async_copy(x_hbm.at[idx_v.at[0]], buf, sem).wait()

    def accumulate(i, buf):
        for u in range(C):
            base = u * K
            node = i * C + u
            accs = [buf[base, pl.ds(c * LANES, LANES)] for c in range(NVEC)]
            cur = [buf[base + 1, pl.ds(c * LANES, LANES)] for c in range(NVEC)]
            for j in range(2, K + 1):
                nxt = (
                    [buf[base + j, pl.ds(c * LANES, LANES)] for c in range(NVEC)]
                    if j < K
                    else None
                )
                accs = [accs[c] + cur[c] for c in range(NVEC)]
                cur = nxt
            for c in range(NVEC):
                out_v[node, pl.ds(c * LANES, LANES)] = accs[c]

    for u in range(NBUF):
        start(u, bufs[u], sems[u])

    def step(h, _):
        for u in range(NBUF):
            i = NBUF * h + u
            wait(bufs[u], sems[u])
            accumulate(i, bufs[u])

            @pl.when(i + NBUF < NCH)
            def _():
                start(i + NBUF, bufs[u], sems[u])

        return 0

    lax.fori_loop(0, NCH // NBUF, step, 0)
    for u in range(NCH % NBUF):
        i = (NCH // NBUF) * NBUF + u
        wait(bufs[u], sems[u])
        accumulate(i, bufs[u])
    pltpu.sync_copy(out_v, out_hbm.at[pl.ds(wid * BPW, BPW)])


_sc_gather_sum = functools.partial(
    pl.kernel,
    out_type=jax.ShapeDtypeStruct((NP, D), jnp.float32),
    mesh=plsc.VectorSubcoreMesh(core_axis_name="c", subcore_axis_name="s"),
    scratch_types=[
        pltpu.VMEM((NCH, CK), jnp.int32),
        pltpu.VMEM((CK, D), jnp.float32),
        pltpu.VMEM((CK, D), jnp.float32),
        pltpu.VMEM((CK, D), jnp.float32),
        pltpu.VMEM((CK, D), jnp.float32),
        pltpu.VMEM((CK, D), jnp.float32),
        pltpu.VMEM((CK, D), jnp.float32),
        pltpu.VMEM((CK, D), jnp.float32),
        pltpu.VMEM((CK, D), jnp.float32),
        pltpu.VMEM((BPW, D), jnp.float32),
        pltpu.SemaphoreType.DMA,
        pltpu.SemaphoreType.DMA,
        pltpu.SemaphoreType.DMA,
        pltpu.SemaphoreType.DMA,
        pltpu.SemaphoreType.DMA,
        pltpu.SemaphoreType.DMA,
        pltpu.SemaphoreType.DMA,
        pltpu.SemaphoreType.DMA,
    ],
)(_sc_body)


def _tc_body(x_ref, s_ref, ws_ref, wn_ref, b_ref, o_ref):
    acc = jnp.dot(x_ref[...], ws_ref[...], preferred_element_type=jnp.float32)
    acc += jnp.dot(s_ref[...], wn_ref[...], preferred_element_type=jnp.float32)
    o_ref[...] = jnp.maximum(acc + b_ref[...], 0.0)


def _tc_dense(X, S, W_self, W_neigh, bias2d):
    bn = 2000
    return pl.pallas_call(
        _tc_body,
        grid=(N // bn,),
        in_specs=[
            pl.BlockSpec((bn, D), lambda i: (i, 0)),
            pl.BlockSpec((bn, D), lambda i: (i, 0)),
            pl.BlockSpec((D, D), lambda i: (0, 0)),
            pl.BlockSpec((D, D), lambda i: (0, 0)),
            pl.BlockSpec((1, D), lambda i: (0, 0)),
        ],
        out_specs=pl.BlockSpec((bn, D), lambda i: (i, 0)),
        out_shape=jax.ShapeDtypeStruct((N, D), jnp.float32),
    )(X, S, W_self, W_neigh, bias2d)


@jax.jit
def kernel(X, neighbors, W_self, W_neigh, bias):
    pad_idx = (jnp.arange((NP - N) * K, dtype=jnp.int32) % N).reshape(NP - N, K)
    nbr_p = jnp.concatenate([neighbors, pad_idx], axis=0)
    nbr2d = nbr_p.reshape(NP * K // CK, CK)
    S = _sc_gather_sum(X, nbr2d)
    return _tc_dense(X, S[:N], W_self, W_neigh, bias.reshape(1, D))


# speedup vs baseline: 1.0429x; 1.0429x over previous
import functools

import jax
import jax.numpy as jnp
from jax import lax
from jax.experimental import pallas as pl
from jax.experimental.pallas import tpu as pltpu
from jax.experimental.pallas import tpu_sc as plsc

N = 10000
K = 32
D = 128
NW = 32
NP = 10240
BPW = NP // NW
C = 1
CK = C * K
NCH = BPW // C
NBUF = 6
LANES = 16
NVEC = D // LANES


def _sc_body(x_hbm, nbr_hbm, out_hbm, idx_v, buf0, buf1, buf2, buf3, buf4, buf5, out_v, sem0, sem1, sem2, sem3, sem4, sem5):
    cid = lax.axis_index("c")
    sid = lax.axis_index("s")
    wid = sid * 2 + cid
    pltpu.sync_copy(nbr_hbm.at[pl.ds(wid * NCH, NCH)], idx_v)

    bufs = (buf0, buf1, buf2, buf3, buf4, buf5)
    sems = (sem0, sem1, sem2, sem3, sem4, sem5)

    def start(i, buf, sem):
        pltpu.async_copy(x_hbm.at[idx_v.at[i]], buf, sem)

    def wait(buf, sem):
        pltpu.make_async_copy(x_hbm.at[idx_v.at[0]], buf, sem).wait()

    def accumulate(i, buf):
        for u in range(C):
            base = u * K
            node = i * C + u
            accs = [buf[base, pl.ds(c * LANES, LANES)] for c in range(NVEC)]
            cur = [buf[base + 1, pl.ds(c * LANES, LANES)] for c in range(NVEC)]
            for j in range(2, K + 1):
                nxt = (
                    [buf[base + j, pl.ds(c * LANES, LANES)] for c in range(NVEC)]
                    if j < K
                    else None
                )
                accs = [accs[c] + cur[c] for c in range(NVEC)]
                cur = nxt
            for c in range(NVEC):
                out_v[node, pl.ds(c * LANES, LANES)] = accs[c]

    for u in range(NBUF):
        start(u, bufs[u], sems[u])

    def step(h, _):
        for u in range(NBUF):
            i = NBUF * h + u
            wait(bufs[u], sems[u])
            accumulate(i, bufs[u])

            @pl.when(i + NBUF < NCH)
            def _():
                start(i + NBUF, bufs[u], sems[u])

        return 0

    lax.fori_loop(0, NCH // NBUF, step, 0)
    for u in range(NCH % NBUF):
        i = (NCH // NBUF) * NBUF + u
        wait(bufs[u], sems[u])
        accumulate(i, bufs[u])
    pltpu.sync_copy(out_v, out_hbm.at[pl.ds(wid * BPW, BPW)])


_sc_gather_sum = functools.partial(
    pl.kernel,
    out_type=jax.ShapeDtypeStruct((NP, D), jnp.float32),
    mesh=plsc.VectorSubcoreMesh(core_axis_name="c", subcore_axis_name="s"),
    scratch_types=[
        pltpu.VMEM((NCH, CK), jnp.int32),
        pltpu.VMEM((CK, D), jnp.float32),
        pltpu.VMEM((CK, D), jnp.float32),
        pltpu.VMEM((CK, D), jnp.float32),
        pltpu.VMEM((CK, D), jnp.float32),
        pltpu.VMEM((CK, D), jnp.float32),
        pltpu.VMEM((CK, D), jnp.float32),
        pltpu.VMEM((BPW, D), jnp.float32),
        pltpu.SemaphoreType.DMA,
        pltpu.SemaphoreType.DMA,
        pltpu.SemaphoreType.DMA,
        pltpu.SemaphoreType.DMA,
        pltpu.SemaphoreType.DMA,
        pltpu.SemaphoreType.DMA,
    ],
)(_sc_body)


def _tc_body(x_ref, s_ref, ws_ref, wn_ref, b_ref, o_ref):
    acc = jnp.dot(x_ref[...], ws_ref[...], preferred_element_type=jnp.float32)
    acc += jnp.dot(s_ref[...], wn_ref[...], preferred_element_type=jnp.float32)
    o_ref[...] = jnp.maximum(acc + b_ref[...], 0.0)


def _tc_dense(X, S, W_self, W_neigh, bias2d):
    bn = 1000
    return pl.pallas_call(
        _tc_body,
        grid=(N // bn,),
        in_specs=[
            pl.BlockSpec((bn, D), lambda i: (i, 0)),
            pl.BlockSpec((bn, D), lambda i: (i, 0)),
            pl.BlockSpec((D, D), lambda i: (0, 0)),
            pl.BlockSpec((D, D), lambda i: (0, 0)),
            pl.BlockSpec((1, D), lambda i: (0, 0)),
        ],
        out_specs=pl.BlockSpec((bn, D), lambda i: (i, 0)),
        out_shape=jax.ShapeDtypeStruct((N, D), jnp.float32),
    )(X, S, W_self, W_neigh, bias2d)


@jax.jit
def kernel(X, neighbors, W_self, W_neigh, bias):
    pad_idx = (jnp.arange((NP - N) * K, dtype=jnp.int32) % N).reshape(NP - N, K)
    nbr_p = jnp.concatenate([neighbors, pad_idx], axis=0)
    nbr2d = nbr_p.reshape(NP * K // CK, CK)
    S = _sc_gather_sum(X, nbr2d)
    return _tc_dense(X, S[:N], W_self, W_neigh, bias.reshape(1, D))


# speedup vs baseline: 1.0725x; 1.0284x over previous
import functools

import jax
import jax.numpy as jnp
from jax import lax
from jax.experimental import pallas as pl
from jax.experimental.pallas import tpu as pltpu
from jax.experimental.pallas import tpu_sc as plsc

N = 10000
K = 32
D = 128
NW = 32
NP = 10240
BPW = NP // NW
C = 1
CK = C * K
NCH = BPW // C
NBUF = 6
LANES = 16
NVEC = D // LANES


def _sc_body(x_hbm, nbr_hbm, out_hbm, idx_v, buf0, buf1, buf2, buf3, buf4, buf5, out_v, sem0, sem1, sem2, sem3, sem4, sem5):
    cid = lax.axis_index("c")
    sid = lax.axis_index("s")
    wid = sid * 2 + cid
    pltpu.sync_copy(nbr_hbm.at[pl.ds(wid * NCH, NCH)], idx_v)

    bufs = (buf0, buf1, buf2, buf3, buf4, buf5)
    sems = (sem0, sem1, sem2, sem3, sem4, sem5)

    def start(i, buf, sem):
        pltpu.async_copy(x_hbm.at[idx_v.at[i]], buf, sem)

    def wait(buf, sem):
        pltpu.make_async_copy(x_hbm.at[idx_v.at[0]], buf, sem).wait()

    def accumulate(i, buf):
        for u in range(C):
            base = u * K
            node = i * C + u
            accs = [buf[base, pl.ds(c * LANES, LANES)] for c in range(NVEC)]
            cur = [buf[base + 1, pl.ds(c * LANES, LANES)] for c in range(NVEC)]
            for j in range(2, K + 1):
                nxt = (
                    [buf[base + j, pl.ds(c * LANES, LANES)] for c in range(NVEC)]
                    if j < K
                    else None
                )
                accs = [accs[c] + cur[c] for c in range(NVEC)]
                cur = nxt
            for c in range(NVEC):
                out_v[node, pl.ds(c * LANES, LANES)] = accs[c]

    for u in range(NBUF):
        start(u, bufs[u], sems[u])

    def step(h, _):
        for u in range(NBUF):
            i = NBUF * h + u
            wait(bufs[u], sems[u])
            accumulate(i, bufs[u])

            @pl.when(i + NBUF < NCH)
            def _():
                start(i + NBUF, bufs[u], sems[u])

        return 0

    lax.fori_loop(0, NCH // NBUF, step, 0)
    for u in range(NCH % NBUF):
        i = (NCH // NBUF) * NBUF + u
        wait(bufs[u], sems[u])
        accumulate(i, bufs[u])
    pltpu.sync_copy(out_v, out_hbm.at[pl.ds(wid * BPW, BPW)])


_sc_gather_sum = functools.partial(
    pl.kernel,
    out_type=jax.ShapeDtypeStruct((NP, D), jnp.float32),
    mesh=plsc.VectorSubcoreMesh(core_axis_name="c", subcore_axis_name="s"),
    scratch_types=[
        pltpu.VMEM((NCH, CK), jnp.int32),
        pltpu.VMEM((CK, D), jnp.float32),
        pltpu.VMEM((CK, D), jnp.float32),
        pltpu.VMEM((CK, D), jnp.float32),
        pltpu.VMEM((CK, D), jnp.float32),
        pltpu.VMEM((CK, D), jnp.float32),
        pltpu.VMEM((CK, D), jnp.float32),
        pltpu.VMEM((BPW, D), jnp.float32),
        pltpu.SemaphoreType.DMA,
        pltpu.SemaphoreType.DMA,
        pltpu.SemaphoreType.DMA,
        pltpu.SemaphoreType.DMA,
        pltpu.SemaphoreType.DMA,
        pltpu.SemaphoreType.DMA,
    ],
)(_sc_body)


def _tc_body(x_ref, s_ref, ws_ref, wn_ref, b_ref, o_ref):
    acc = jnp.dot(x_ref[...], ws_ref[...], preferred_element_type=jnp.float32)
    acc += jnp.dot(s_ref[...], wn_ref[...], preferred_element_type=jnp.float32)
    o_ref[...] = jnp.maximum(acc + b_ref[...], 0.0)


def _tc_dense(X, S, W_self, W_neigh, bias2d):
    bn = 2000
    return pl.pallas_call(
        _tc_body,
        grid=(N // bn,),
        in_specs=[
            pl.BlockSpec((bn, D), lambda i: (i, 0)),
            pl.BlockSpec((bn, D), lambda i: (i, 0)),
            pl.BlockSpec((D, D), lambda i: (0, 0)),
            pl.BlockSpec((D, D), lambda i: (0, 0)),
            pl.BlockSpec((1, D), lambda i: (0, 0)),
        ],
        out_specs=pl.BlockSpec((bn, D), lambda i: (i, 0)),
        out_shape=jax.ShapeDtypeStruct((N, D), jnp.float32),
    )(X, S, W_self, W_neigh, bias2d)


@jax.jit
def kernel(X, neighbors, W_self, W_neigh, bias):
    pad_idx = (jnp.arange((NP - N) * K, dtype=jnp.int32) % N).reshape(NP - N, K)
    nbr_p = jnp.concatenate([neighbors, pad_idx], axis=0)
    nbr2d = nbr_p.reshape(NP * K // CK, CK)
    S = _sc_gather_sum(X, nbr2d)
    return _tc_dense(X, S[:N], W_self, W_neigh, bias.reshape(1, D))


# speedup vs baseline: 1.1809x; 1.1011x over previous
import functools

import jax
import jax.numpy as jnp
from jax import lax
from jax.experimental import pallas as pl
from jax.experimental.pallas import tpu as pltpu
from jax.experimental.pallas import tpu_sc as plsc

N = 10000
K = 32
D = 128
NW = 32
NP = 10240
BPW = NP // NW
C = 1
CK = C * K
NCH = BPW // C
NBUF = 5
LANES = 16
NVEC = D // LANES


def _sc_body(x_hbm, nbr_hbm, out_hbm, idx_v, buf0, buf1, buf2, buf3, buf4, buf5, out_v, sem0, sem1, sem2, sem3, sem4, sem5):
    cid = lax.axis_index("c")
    sid = lax.axis_index("s")
    wid = sid * 2 + cid
    pltpu.sync_copy(nbr_hbm.at[pl.ds(wid * NCH, NCH)], idx_v)

    bufs = (buf0, buf1, buf2, buf3, buf4, buf5)
    sems = (sem0, sem1, sem2, sem3, sem4, sem5)

    def start(i, buf, sem):
        pltpu.async_copy(x_hbm.at[idx_v.at[i]], buf, sem)

    def wait(buf, sem):
        pltpu.make_async_copy(x_hbm.at[idx_v.at[0]], buf, sem).wait()

    def accumulate(i, buf):
        for u in range(C):
            base = u * K
            node = i * C + u
            accs = [buf[base, pl.ds(c * LANES, LANES)] for c in range(NVEC)]
            cur = [buf[base + 1, pl.ds(c * LANES, LANES)] for c in range(NVEC)]
            for j in range(2, K + 1):
                nxt = (
                    [buf[base + j, pl.ds(c * LANES, LANES)] for c in range(NVEC)]
                    if j < K
                    else None
                )
                accs = [accs[c] + cur[c] for c in range(NVEC)]
                cur = nxt
            for c in range(NVEC):
                out_v[node, pl.ds(c * LANES, LANES)] = accs[c]

    for u in range(NBUF):
        start(u, bufs[u], sems[u])

    def step(h, _):
        for u in range(NBUF):
            i = NBUF * h + u
            wait(bufs[u], sems[u])
            accumulate(i, bufs[u])

            @pl.when(i + NBUF < NCH)
            def _():
                start(i + NBUF, bufs[u], sems[u])

        return 0

    lax.fori_loop(0, NCH // NBUF, step, 0)
    for u in range(NCH % NBUF):
        i = (NCH // NBUF) * NBUF + u
        wait(bufs[u], sems[u])
        accumulate(i, bufs[u])
    pltpu.sync_copy(out_v, out_hbm.at[pl.ds(wid * BPW, BPW)])


_sc_gather_sum = functools.partial(
    pl.kernel,
    out_type=jax.ShapeDtypeStruct((NP, D), jnp.float32),
    mesh=plsc.VectorSubcoreMesh(core_axis_name="c", subcore_axis_name="s"),
    scratch_types=[
        pltpu.VMEM((NCH, CK), jnp.int32),
        pltpu.VMEM((CK, D), jnp.float32),
        pltpu.VMEM((CK, D), jnp.float32),
        pltpu.VMEM((CK, D), jnp.float32),
        pltpu.VMEM((CK, D), jnp.float32),
        pltpu.VMEM((CK, D), jnp.float32),
        pltpu.VMEM((CK, D), jnp.float32),
        pltpu.VMEM((BPW, D), jnp.float32),
        pltpu.SemaphoreType.DMA,
        pltpu.SemaphoreType.DMA,
        pltpu.SemaphoreType.DMA,
        pltpu.SemaphoreType.DMA,
        pltpu.SemaphoreType.DMA,
        pltpu.SemaphoreType.DMA,
    ],
)(_sc_body)


def _tc_body(x_ref, s_ref, ws_ref, wn_ref, b_ref, o_ref):
    acc = jnp.dot(x_ref[...], ws_ref[...], preferred_element_type=jnp.float32)
    acc += jnp.dot(s_ref[...], wn_ref[...], preferred_element_type=jnp.float32)
    o_ref[...] = jnp.maximum(acc + b_ref[...], 0.0)


def _tc_dense(X, S, W_self, W_neigh, bias2d):
    bn = 2000
    return pl.pallas_call(
        _tc_body,
        grid=(N // bn,),
        in_specs=[
            pl.BlockSpec((bn, D), lambda i: (i, 0)),
            pl.BlockSpec((bn, D), lambda i: (i, 0)),
            pl.BlockSpec((D, D), lambda i: (0, 0)),
            pl.BlockSpec((D, D), lambda i: (0, 0)),
            pl.BlockSpec((1, D), lambda i: (0, 0)),
        ],
        out_specs=pl.BlockSpec((bn, D), lambda i: (i, 0)),
        out_shape=jax.ShapeDtypeStruct((N, D), jnp.float32),
    )(X, S, W_self, W_neigh, bias2d)


@jax.jit
def kernel(X, neighbors, W_self, W_neigh, bias):
    pad_idx = (jnp.arange((NP - N) * K, dtype=jnp.int32) % N).reshape(NP - N, K)
    nbr_p = jnp.concatenate([neighbors, pad_idx], axis=0)
    nbr2d = nbr_p.reshape(NP * K // CK, CK)
    S = _sc_gather_sum(X, nbr2d)
    return _tc_dense(X, S[:N], W_self, W_neigh, bias.reshape(1, D))
